# baseline (speedup 1.0000x reference)
import functools

import jax
import jax.numpy as jnp
import numpy as np
from jax import lax
from jax.experimental import pallas as pl
from jax.experimental.pallas import tpu as pltpu
from jax.experimental.pallas import tpu_sc as plsc

N, E, FIN, H, C = 10000, 320000, 128, 64, 2
NPAD = 10240
NW = 32
ROWS_PER_TILE = NPAD // 16

K = 128
NSTEP = 79
EPT = K * NSTEP
EPAD = EPT * NW

PERM = np.array([32 * (j // 32) + (j % 32 % 2) * 16 + (j % 32) // 2
                 for j in range(H)], dtype=np.int32)


@functools.cache
def _make_deg_kernel():
    return functools.partial(
        pl.kernel,
        out_type=jax.ShapeDtypeStruct((2, NPAD), jnp.float32),
        mesh=plsc.VectorSubcoreMesh(core_axis_name="c", subcore_axis_name="s"),
        compiler_params=pltpu.CompilerParams(use_tc_tiling_on_sc=False,
                                             needs_layout_passes=False),
        scratch_types=[
            pltpu.VMEM((K,), jnp.int32),
            pltpu.VMEM((K,), jnp.int32),
            pltpu.VMEM((EPT,), jnp.float32),
            pltpu.VMEM((ROWS_PER_TILE,), jnp.float32),
            pltpu.VMEM_SHARED((NPAD,), jnp.float32),
            pltpu.SemaphoreType.DMA,
            pltpu.SemaphoreType.DMA,
        ],
    )(_deg_body)


def _deg_body(dst_hbm, ew_hbm, degp_hbm, dstbA, dstbB, ewv, zb, acc_sh,
              dsemA, dsemB):
    cid = lax.axis_index("c")
    sid = lax.axis_index("s")
    wid = cid * 16 + sid

    def zrow(r, carry):
        zb[pl.ds(r * 16, 16)] = jnp.zeros((16,), jnp.float32)
        return carry

    lax.fori_loop(0, ROWS_PER_TILE // 16, zrow, 0)
    pltpu.sync_copy(zb, acc_sh.at[pl.ds(sid * ROWS_PER_TILE, ROWS_PER_TILE)])
    plsc.subcore_barrier()

    gbase = wid * EPT
    pltpu.sync_copy(ew_hbm.at[pl.ds(gbase, EPT)], ewv)

    def issue(off, dstb, dsem):
        pltpu.async_copy(dst_hbm.at[pl.ds(gbase + off, K)], dstb, dsem)

    def drain(off, dstb, dsem):
        pltpu.make_async_copy(dst_hbm.at[pl.ds(gbase, K)], dstb, dsem).wait()
        pltpu.sync_copy(ewv.at[pl.ds(off, K)], acc_sh.at[dstb], add=True)

    issue(0, dstbA, dsemA)

    def pair(t, carry):
        offA = (2 * t) * K
        issue(offA + K, dstbB, dsemB)
        drain(offA, dstbA, dsemA)
        issue(offA + 2 * K, dstbA, dsemA)
        drain(offA + K, dstbB, dsemB)
        return carry

    lax.fori_loop(0, (NSTEP - 1) // 2, pair, 0)
    drain((NSTEP - 1) * K, dstbA, dsemA)
    plsc.subcore_barrier()
    pltpu.sync_copy(
        acc_sh.at[pl.ds(sid * ROWS_PER_TILE, ROWS_PER_TILE)],
        degp_hbm.at[cid, pl.ds(sid * ROWS_PER_TILE, ROWS_PER_TILE)],
    )


@functools.cache
def _make_scatter_kernel():
    return functools.partial(
        pl.kernel,
        out_type=jax.ShapeDtypeStruct((2, NPAD, H), jnp.float32),
        mesh=plsc.VectorSubcoreMesh(core_axis_name="c", subcore_axis_name="s"),
        compiler_params=pltpu.CompilerParams(use_tc_tiling_on_sc=False,
                                             needs_layout_passes=False),
        scratch_types=[
            pltpu.VMEM((EPT,), jnp.int32),
            pltpu.VMEM((EPT,), jnp.float32),
            pltpu.VMEM((K,), jnp.int32),
            pltpu.VMEM((K,), jnp.int32),
            pltpu.VMEM((K, H), jnp.float32),
            pltpu.VMEM((K, H), jnp.float32),
            pltpu.VMEM((64, H), jnp.float32),
            pltpu.VMEM_SHARED((NPAD, H), jnp.float32),
            pltpu.SemaphoreType.DMA,
            pltpu.SemaphoreType.DMA,
            pltpu.SemaphoreType.DMA,
            pltpu.SemaphoreType.DMA,
        ],
    )(_scatter_body)


def _scatter_body(y_hbm, src_hbm, dst_hbm, ew_hbm, acc_hbm,
                  srcb, ewv, dstbA, dstbB, rowsA, rowsB, zb, acc_sh,
                  gsemA, gsemB, dsemA, dsemB):
    cid = lax.axis_index("c")
    sid = lax.axis_index("s")
    wid = cid * 16 + sid

    def zrow(r, carry):
        for f in range(H // 16):
            zb[r, pl.ds(f * 16, 16)] = jnp.zeros((16,), jnp.float32)
        return carry

    lax.fori_loop(0, 64, zrow, 0)
    for t in range(ROWS_PER_TILE // 64):
        pltpu.sync_copy(zb, acc_sh.at[pl.ds(sid * ROWS_PER_TILE + t * 64, 64)])
    plsc.subcore_barrier()

    gbase = wid * EPT
    pltpu.sync_copy(src_hbm.at[pl.ds(gbase, EPT)], srcb)
    pltpu.sync_copy(ew_hbm.at[pl.ds(gbase, EPT)], ewv)

    def issue(off, rows, dstb, gsem, dsem):
        pltpu.async_copy(y_hbm.at[srcb.at[pl.ds(off, K)]], rows, gsem)
        pltpu.async_copy(dst_hbm.at[pl.ds(gbase + off, K)], dstb, dsem)

    def scale(off, rows):
        for j in range(K):
            s = plsc.load_gather(ewv, [jnp.full((16,), off + j, jnp.int32)])
            for f in range(H // 16):
                rows[j, pl.ds(f * 16, 16)] = rows[j, pl.ds(f * 16, 16)] * s

    def drain(off, rows, dstb, gsem, dsem):
        pltpu.make_async_copy(y_hbm.at[srcb.at[pl.ds(0, K)]], rows,
                              gsem).wait()
        scale(off, rows)
        pltpu.make_async_copy(dst_hbm.at[pl.ds(gbase, K)], dstb, dsem).wait()
        pltpu.sync_copy(rows, acc_sh.at[dstb], add=True)

    issue(0, rowsA, dstbA, gsemA, dsemA)

    def pair(t, carry):
        offA = (2 * t) * K
        issue(offA + K, rowsB, dstbB, gsemB, dsemB)
        drain(offA, rowsA, dstbA, gsemA, dsemA)
        issue(offA + 2 * K, rowsA, dstbA, gsemA, dsemA)
        drain(offA + K, rowsB, dstbB, gsemB, dsemB)
        return carry

    lax.fori_loop(0, (NSTEP - 1) // 2, pair, 0)
    drain((NSTEP - 1) * K, rowsA, dstbA, gsemA, dsemA)

    plsc.subcore_barrier()
    pltpu.sync_copy(
        acc_sh.at[pl.ds(sid * ROWS_PER_TILE, ROWS_PER_TILE)],
        acc_hbm.at[cid, pl.ds(sid * ROWS_PER_TILE, ROWS_PER_TILE)],
    )


BLK = 1024
GRID = NPAD // BLK


def _lrelu(x):
    return jnp.where(x > 0, x, 0.01 * x)


def _dis_block(degp):
    deg = degp[0] + degp[1] + 1.0
    return lax.rsqrt(deg)[:, None]


def _mm1_body(x_ref, w_ref, degp_ref, xw_ref, y_ref):
    xw = jnp.dot(x_ref[...], w_ref[...], preferred_element_type=jnp.float32)
    dis = _dis_block(degp_ref[...])
    xw_ref[...] = xw
    y_ref[...] = xw * dis


def _mm1(X_pad, W1, degp):
    return pl.pallas_call(
        _mm1_body,
        grid=(GRID,),
        in_specs=[
            pl.BlockSpec((BLK, FIN), lambda i: (i, 0)),
            pl.BlockSpec((FIN, H), lambda i: (0, 0)),
            pl.BlockSpec((2, BLK), lambda i: (0, i)),
        ],
        out_specs=[pl.BlockSpec((BLK, H), lambda i: (i, 0))] * 2,
        out_shape=[jax.ShapeDtypeStruct((NPAD, H), jnp.float32)] * 2,
    )(X_pad, W1, degp)


def _mm2_body(accp_ref, xw1_ref, degp_ref, b_ref, w_ref, xw2_ref, y2_ref):
    acc = accp_ref[0] + accp_ref[1]
    dis = _dis_block(degp_ref[...])
    h = dis * acc + (dis * dis) * xw1_ref[...] + b_ref[...]
    h = _lrelu(h)
    xw2 = jnp.dot(h, w_ref[...], preferred_element_type=jnp.float32)
    xw2_ref[...] = xw2
    y2_ref[...] = xw2 * dis


def _mm2(accp, xw1, degp, b1, W2):
    return pl.pallas_call(
        _mm2_body,
        grid=(GRID,),
        in_specs=[
            pl.BlockSpec((2, BLK, H), lambda i: (0, i, 0)),
            pl.BlockSpec((BLK, H), lambda i: (i, 0)),
            pl.BlockSpec((2, BLK), lambda i: (0, i)),
            pl.BlockSpec((1, H), lambda i: (0, 0)),
            pl.BlockSpec((H, H), lambda i: (0, 0)),
        ],
        out_specs=[pl.BlockSpec((BLK, H), lambda i: (i, 0))] * 2,
        out_shape=[jax.ShapeDtypeStruct((NPAD, H), jnp.float32)] * 2,
    )(accp, xw1, degp, b1, W2)


def _head_body(accp_ref, xw2_ref, degp_ref, b2_ref, wm1_ref, bm1_ref,
               wm2_ref, bm2_ref, out_ref):
    acc = accp_ref[0] + accp_ref[1]
    dis = _dis_block(degp_ref[...])
    h = dis * acc + (dis * dis) * xw2_ref[...] + b2_ref[...]
    h = _lrelu(h)
    h = _lrelu(jnp.dot(h, wm1_ref[...], preferred_element_type=jnp.float32)
               + bm1_ref[...])
    logits = jnp.dot(h, wm2_ref[...], preferred_element_type=jnp.float32) \
        + bm2_ref[...]
    m = jnp.max(logits, axis=-1, keepdims=True)
    e = jnp.exp(logits - m)
    out_ref[...] = e / jnp.sum(e, axis=-1, keepdims=True)


def _head(accp, xw2, degp, b2, Wm1, bm1, Wm2, bm2):
    return pl.pallas_call(
        _head_body,
        grid=(GRID,),
        in_specs=[
            pl.BlockSpec((2, BLK, H), lambda i: (0, i, 0)),
            pl.BlockSpec((BLK, H), lambda i: (i, 0)),
            pl.BlockSpec((2, BLK), lambda i: (0, i)),
            pl.BlockSpec((1, H), lambda i: (0, 0)),
            pl.BlockSpec((H, H), lambda i: (0, 0)),
            pl.BlockSpec((1, H), lambda i: (0, 0)),
            pl.BlockSpec((H, C), lambda i: (0, 0)),
            pl.BlockSpec((1, C), lambda i: (0, 0)),
        ],
        out_specs=pl.BlockSpec((BLK, C), lambda i: (i, 0)),
        out_shape=jax.ShapeDtypeStruct((NPAD, C), jnp.float32),
    )(accp, xw2, degp, b2, Wm1, bm1, Wm2, bm2)


def kernel(X, edge_index, edge_weight, W1, b1, W2, b2, Wm1, bm1, Wm2, bm2):
    src = jnp.pad(edge_index[0], (0, EPAD - E))
    dst = jnp.pad(edge_index[1], (0, EPAD - E))
    ew = jnp.pad(edge_weight, (0, EPAD - E))
    X_pad = jnp.pad(X, ((0, NPAD - N), (0, 0)))

    deg_k = _make_deg_kernel()
    scat_k = _make_scatter_kernel()
    degp = deg_k(dst, ew)
    xw1, y1 = _mm1(X_pad, W1, degp)
    acc1 = scat_k(y1, src, dst, ew)
    xw2, y2 = _mm2(acc1, xw1, degp, b1.reshape(1, H), W2)
    acc2 = scat_k(y2, src, dst, ew)
    out = _head(acc2, xw2, degp, b2.reshape(1, H), Wm1, bm1.reshape(1, H),
                Wm2, bm2.reshape(1, C))
    return out[:N]


# speedup vs baseline: 1.0991x; 1.0991x over previous
import functools

import jax
import jax.numpy as jnp
import numpy as np
from jax import lax
from jax.experimental import pallas as pl
from jax.experimental.pallas import tpu as pltpu
from jax.experimental.pallas import tpu_sc as plsc

N, E, FIN, H, C = 10000, 320000, 128, 64, 2
NPAD = 10240
NW = 32
ROWS_PER_TILE = NPAD // 16

K = 80
NSTEP = 125
EPT = K * NSTEP
EPAD = EPT * NW

PERM = np.array([32 * (j // 32) + (j % 32 % 2) * 16 + (j % 32) // 2
                 for j in range(H)], dtype=np.int32)


@functools.cache
def _make_deg_kernel():
    return functools.partial(
        pl.kernel,
        out_type=jax.ShapeDtypeStruct((2, NPAD), jnp.float32),
        mesh=plsc.VectorSubcoreMesh(core_axis_name="c", subcore_axis_name="s"),
        compiler_params=pltpu.CompilerParams(use_tc_tiling_on_sc=False,
                                             needs_layout_passes=False),
        scratch_types=[
            pltpu.VMEM((K,), jnp.int32),
            pltpu.VMEM((K,), jnp.int32),
            pltpu.VMEM((EPT,), jnp.float32),
            pltpu.VMEM((ROWS_PER_TILE,), jnp.float32),
            pltpu.VMEM_SHARED((NPAD,), jnp.float32),
            pltpu.SemaphoreType.DMA,
            pltpu.SemaphoreType.DMA,
        ],
    )(_deg_body)


def _deg_body(dst_hbm, ew_hbm, degp_hbm, dstbA, dstbB, ewv, zb, acc_sh,
              dsemA, dsemB):
    cid = lax.axis_index("c")
    sid = lax.axis_index("s")
    wid = cid * 16 + sid

    def zrow(r, carry):
        zb[pl.ds(r * 16, 16)] = jnp.zeros((16,), jnp.float32)
        return carry

    lax.fori_loop(0, ROWS_PER_TILE // 16, zrow, 0)
    pltpu.sync_copy(zb, acc_sh.at[pl.ds(sid * ROWS_PER_TILE, ROWS_PER_TILE)])
    plsc.subcore_barrier()

    gbase = wid * EPT
    pltpu.sync_copy(ew_hbm.at[pl.ds(gbase, EPT)], ewv)

    def issue(off, dstb, dsem):
        pltpu.async_copy(dst_hbm.at[pl.ds(gbase + off, K)], dstb, dsem)

    def drain(off, dstb, dsem):
        pltpu.make_async_copy(dst_hbm.at[pl.ds(gbase, K)], dstb, dsem).wait()
        pltpu.sync_copy(ewv.at[pl.ds(off, K)], acc_sh.at[dstb], add=True)

    issue(0, dstbA, dsemA)

    def pair(t, carry):
        offA = (2 * t) * K
        issue(offA + K, dstbB, dsemB)
        drain(offA, dstbA, dsemA)
        issue(offA + 2 * K, dstbA, dsemA)
        drain(offA + K, dstbB, dsemB)
        return carry

    lax.fori_loop(0, (NSTEP - 1) // 2, pair, 0)
    drain((NSTEP - 1) * K, dstbA, dsemA)
    plsc.subcore_barrier()
    pltpu.sync_copy(
        acc_sh.at[pl.ds(sid * ROWS_PER_TILE, ROWS_PER_TILE)],
        degp_hbm.at[cid, pl.ds(sid * ROWS_PER_TILE, ROWS_PER_TILE)],
    )


@functools.cache
def _make_scatter_kernel():
    return functools.partial(
        pl.kernel,
        out_type=jax.ShapeDtypeStruct((2, NPAD, H), jnp.float32),
        mesh=plsc.VectorSubcoreMesh(core_axis_name="c", subcore_axis_name="s"),
        compiler_params=pltpu.CompilerParams(use_tc_tiling_on_sc=False,
                                             needs_layout_passes=False),
        scratch_types=[
            pltpu.VMEM((EPT,), jnp.int32),
            pltpu.VMEM((EPT,), jnp.float32),
            pltpu.VMEM((K,), jnp.int32),
            pltpu.VMEM((K,), jnp.int32),
            pltpu.VMEM((K, H), jnp.bfloat16),
            pltpu.VMEM((K, H), jnp.bfloat16),
            pltpu.VMEM((K, H), jnp.float32),
            pltpu.VMEM((K, H), jnp.float32),
            pltpu.VMEM((64, H), jnp.float32),
            pltpu.VMEM_SHARED((NPAD, H), jnp.float32),
            pltpu.SemaphoreType.DMA,
            pltpu.SemaphoreType.DMA,
            pltpu.SemaphoreType.DMA,
            pltpu.SemaphoreType.DMA,
        ],
    )(_scatter_body)


def _scatter_body(y_hbm, src_hbm, dst_hbm, ew_hbm, acc_hbm,
                  srcb, ewv, dstbA, dstbB, rbfA, rbfB, rfA, rfB, zb, acc_sh,
                  gsemA, gsemB, dsemA, dsemB):
    cid = lax.axis_index("c")
    sid = lax.axis_index("s")
    wid = cid * 16 + sid

    def zrow(r, carry):
        for f in range(H // 16):
            zb[r, pl.ds(f * 16, 16)] = jnp.zeros((16,), jnp.float32)
        return carry

    lax.fori_loop(0, 64, zrow, 0)
    for t in range(ROWS_PER_TILE // 64):
        pltpu.sync_copy(zb, acc_sh.at[pl.ds(sid * ROWS_PER_TILE + t * 64, 64)])
    plsc.subcore_barrier()

    gbase = wid * EPT
    pltpu.sync_copy(src_hbm.at[pl.ds(gbase, EPT)], srcb)
    pltpu.sync_copy(ew_hbm.at[pl.ds(gbase, EPT)], ewv)

    def issue(off, rbf, dstb, gsem, dsem):
        pltpu.async_copy(y_hbm.at[srcb.at[pl.ds(off, K)]], rbf, gsem)
        pltpu.async_copy(dst_hbm.at[pl.ds(gbase + off, K)], dstb, dsem)

    def scale(off, rbf, rf):
        for j in range(K):
            s = plsc.load_gather(ewv, [jnp.full((16,), off + j, jnp.int32)])
            for g in range(H // 32):
                v = rbf[j, pl.ds(32 * g, 32)]
                a, b = plsc.unpack(v, format=plsc.PackFormat.INTERLEAVED)
                rf[j, pl.ds(32 * g, 16)] = a * s
                rf[j, pl.ds(32 * g + 16, 16)] = b * s

    def drain(off, rbf, rf, dstb, gsem, dsem):
        pltpu.make_async_copy(y_hbm.at[srcb.at[pl.ds(0, K)]], rbf,
                              gsem).wait()
        scale(off, rbf, rf)
        pltpu.make_async_copy(dst_hbm.at[pl.ds(gbase, K)], dstb, dsem).wait()
        pltpu.sync_copy(rf, acc_sh.at[dstb], add=True)

    issue(0, rbfA, dstbA, gsemA, dsemA)

    def pair(t, carry):
        offA = (2 * t) * K
        issue(offA + K, rbfB, dstbB, gsemB, dsemB)
        drain(offA, rbfA, rfA, dstbA, gsemA, dsemA)
        issue(offA + 2 * K, rbfA, dstbA, gsemA, dsemA)
        drain(offA + K, rbfB, rfB, dstbB, gsemB, dsemB)
        return carry

    lax.fori_loop(0, (NSTEP - 1) // 2, pair, 0)
    drain((NSTEP - 1) * K, rbfA, rfA, dstbA, gsemA, dsemA)

    plsc.subcore_barrier()
    pltpu.sync_copy(
        acc_sh.at[pl.ds(sid * ROWS_PER_TILE, ROWS_PER_TILE)],
        acc_hbm.at[cid, pl.ds(sid * ROWS_PER_TILE, ROWS_PER_TILE)],
    )


BLK = 1024
GRID = NPAD // BLK


def _lrelu(x):
    return jnp.where(x > 0, x, 0.01 * x)


def _dis_block(degp):
    deg = degp[0] + degp[1] + 1.0
    return lax.rsqrt(deg)[:, None]


def _mm1_body(x_ref, w_ref, wp_ref, degp_ref, xw_ref, y_ref):
    xw = jnp.dot(x_ref[...], w_ref[...], preferred_element_type=jnp.float32)
    xwp = jnp.dot(x_ref[...], wp_ref[...], preferred_element_type=jnp.float32)
    dis = _dis_block(degp_ref[...])
    xw_ref[...] = xw
    y_ref[...] = (xwp * dis).astype(jnp.bfloat16)


def _mm1(X_pad, W1, W1p, degp):
    return pl.pallas_call(
        _mm1_body,
        grid=(GRID,),
        in_specs=[
            pl.BlockSpec((BLK, FIN), lambda i: (i, 0)),
            pl.BlockSpec((FIN, H), lambda i: (0, 0)),
            pl.BlockSpec((FIN, H), lambda i: (0, 0)),
            pl.BlockSpec((2, BLK), lambda i: (0, i)),
        ],
        out_specs=[pl.BlockSpec((BLK, H), lambda i: (i, 0))] * 2,
        out_shape=[jax.ShapeDtypeStruct((NPAD, H), jnp.float32),
                   jax.ShapeDtypeStruct((NPAD, H), jnp.bfloat16)],
    )(X_pad, W1, W1p, degp)


def _mm2_body(accp_ref, xw1_ref, degp_ref, b_ref, w_ref, wp_ref,
              xw2_ref, y2_ref):
    acc = accp_ref[0] + accp_ref[1]
    dis = _dis_block(degp_ref[...])
    h = dis * acc + (dis * dis) * xw1_ref[...] + b_ref[...]
    h = _lrelu(h)
    xw2 = jnp.dot(h, w_ref[...], preferred_element_type=jnp.float32)
    xw2p = jnp.dot(h, wp_ref[...], preferred_element_type=jnp.float32)
    xw2_ref[...] = xw2
    y2_ref[...] = (xw2p * dis).astype(jnp.bfloat16)


def _mm2(accp, xw1, degp, b1, W2, W2p):
    return pl.pallas_call(
        _mm2_body,
        grid=(GRID,),
        in_specs=[
            pl.BlockSpec((2, BLK, H), lambda i: (0, i, 0)),
            pl.BlockSpec((BLK, H), lambda i: (i, 0)),
            pl.BlockSpec((2, BLK), lambda i: (0, i)),
            pl.BlockSpec((1, H), lambda i: (0, 0)),
            pl.BlockSpec((H, H), lambda i: (0, 0)),
            pl.BlockSpec((H, H), lambda i: (0, 0)),
        ],
        out_specs=[pl.BlockSpec((BLK, H), lambda i: (i, 0))] * 2,
        out_shape=[jax.ShapeDtypeStruct((NPAD, H), jnp.float32),
                   jax.ShapeDtypeStruct((NPAD, H), jnp.bfloat16)],
    )(accp, xw1, degp, b1, W2, W2p)


def _head_body(accp_ref, xw2_ref, degp_ref, b2_ref, wm1_ref, bm1_ref,
               wm2_ref, bm2_ref, out_ref):
    acc = accp_ref[0] + accp_ref[1]
    dis = _dis_block(degp_ref[...])
    h = dis * acc + (dis * dis) * xw2_ref[...] + b2_ref[...]
    h = _lrelu(h)
    h = _lrelu(jnp.dot(h, wm1_ref[...], preferred_element_type=jnp.float32)
               + bm1_ref[...])
    logits = jnp.dot(h, wm2_ref[...], preferred_element_type=jnp.float32) \
        + bm2_ref[...]
    m = jnp.max(logits, axis=-1, keepdims=True)
    e = jnp.exp(logits - m)
    out_ref[...] = e / jnp.sum(e, axis=-1, keepdims=True)


def _head(accp, xw2, degp, b2, Wm1, bm1, Wm2, bm2):
    return pl.pallas_call(
        _head_body,
        grid=(GRID,),
        in_specs=[
            pl.BlockSpec((2, BLK, H), lambda i: (0, i, 0)),
            pl.BlockSpec((BLK, H), lambda i: (i, 0)),
            pl.BlockSpec((2, BLK), lambda i: (0, i)),
            pl.BlockSpec((1, H), lambda i: (0, 0)),
            pl.BlockSpec((H, H), lambda i: (0, 0)),
            pl.BlockSpec((1, H), lambda i: (0, 0)),
            pl.BlockSpec((H, C), lambda i: (0, 0)),
            pl.BlockSpec((1, C), lambda i: (0, 0)),
        ],
        out_specs=pl.BlockSpec((BLK, C), lambda i: (i, 0)),
        out_shape=jax.ShapeDtypeStruct((NPAD, C), jnp.float32),
    )(accp, xw2, degp, b2, Wm1, bm1, Wm2, bm2)


def kernel(X, edge_index, edge_weight, W1, b1, W2, b2, Wm1, bm1, Wm2, bm2):
    src = jnp.pad(edge_index[0], (0, EPAD - E))
    dst = jnp.pad(edge_index[1], (0, EPAD - E))
    ew = jnp.pad(edge_weight, (0, EPAD - E))
    X_pad = jnp.pad(X, ((0, NPAD - N), (0, 0)))
    W1p = W1[:, PERM]
    W2p = W2[:, PERM]

    deg_k = _make_deg_kernel()
    scat_k = _make_scatter_kernel()
    degp = deg_k(dst, ew)
    xw1, y1 = _mm1(X_pad, W1, W1p, degp)
    acc1 = scat_k(y1, src, dst, ew)
    xw2, y2 = _mm2(acc1, xw1, degp, b1.reshape(1, H), W2, W2p)
    acc2 = scat_k(y2, src, dst, ew)
    out = _head(acc2, xw2, degp, b2.reshape(1, H), Wm1, bm1.reshape(1, H),
                Wm2, bm2.reshape(1, C))
    return out[:N]


# speedup vs baseline: 1.1544x; 1.0503x over previous
import functools

import jax
import jax.numpy as jnp
from jax import lax
from jax.experimental import pallas as pl
from jax.experimental.pallas import tpu as pltpu
from jax.experimental.pallas import tpu_sc as plsc

N, E, FIN, H, C = 10000, 320000, 128, 64, 2
NPAD = 10240
NW = 32
ROWS_PER_TILE = NPAD // 16

K = 80
NSTEP = 125
EPT = K * NSTEP

KD = 128
NSTEPD = 79
EPTD = KD * NSTEPD
EPAD = EPTD * NW


@functools.cache
def _make_deg_kernel():
    return functools.partial(
        pl.kernel,
        out_type=jax.ShapeDtypeStruct((2, NPAD), jnp.float32),
        mesh=plsc.VectorSubcoreMesh(core_axis_name="c", subcore_axis_name="s"),
        compiler_params=pltpu.CompilerParams(use_tc_tiling_on_sc=False,
                                             needs_layout_passes=False),
        scratch_types=[
            pltpu.VMEM((KD,), jnp.int32),
            pltpu.VMEM((KD,), jnp.int32),
            pltpu.VMEM((EPTD,), jnp.float32),
            pltpu.VMEM((ROWS_PER_TILE,), jnp.float32),
            pltpu.VMEM_SHARED((NPAD,), jnp.float32),
            pltpu.SemaphoreType.DMA,
            pltpu.SemaphoreType.DMA,
        ],
    )(_deg_body)


def _deg_body(dst_hbm, ew_hbm, degp_hbm, dstbA, dstbB, ewv, zb, acc_sh,
              dsemA, dsemB):
    cid = lax.axis_index("c")
    sid = lax.axis_index("s")
    wid = cid * 16 + sid

    def zrow(r, carry):
        zb[pl.ds(r * 16, 16)] = jnp.zeros((16,), jnp.float32)
        return carry

    lax.fori_loop(0, ROWS_PER_TILE // 16, zrow, 0)
    pltpu.sync_copy(zb, acc_sh.at[pl.ds(sid * ROWS_PER_TILE, ROWS_PER_TILE)])
    plsc.subcore_barrier()

    gbase = wid * EPTD
    pltpu.sync_copy(ew_hbm.at[pl.ds(gbase, EPTD)], ewv)

    def issue(off, dstb, dsem):
        pltpu.async_copy(dst_hbm.at[pl.ds(gbase + off, KD)], dstb, dsem)

    def drain(off, dstb, dsem):
        pltpu.make_async_copy(dst_hbm.at[pl.ds(gbase, KD)], dstb, dsem).wait()
        pltpu.sync_copy(ewv.at[pl.ds(off, KD)], acc_sh.at[dstb], add=True)

    issue(0, dstbA, dsemA)

    def pair(t, carry):
        offA = (2 * t) * KD
        issue(offA + KD, dstbB, dsemB)
        drain(offA, dstbA, dsemA)
        issue(offA + 2 * KD, dstbA, dsemA)
        drain(offA + KD, dstbB, dsemB)
        return carry

    lax.fori_loop(0, (NSTEPD - 1) // 2, pair, 0)
    drain((NSTEPD - 1) * KD, dstbA, dsemA)
    plsc.subcore_barrier()
    pltpu.sync_copy(
        acc_sh.at[pl.ds(sid * ROWS_PER_TILE, ROWS_PER_TILE)],
        degp_hbm.at[cid, pl.ds(sid * ROWS_PER_TILE, ROWS_PER_TILE)],
    )


@functools.cache
def _make_scatter_kernel():
    return functools.partial(
        pl.kernel,
        out_type=jax.ShapeDtypeStruct((2, NPAD, H), jnp.float32),
        mesh=plsc.VectorSubcoreMesh(core_axis_name="c", subcore_axis_name="s"),
        compiler_params=pltpu.CompilerParams(use_tc_tiling_on_sc=False,
                                             needs_layout_passes=False),
        scratch_types=[
            pltpu.VMEM((EPT,), jnp.int32),
            pltpu.VMEM((EPT,), jnp.float32),
            pltpu.VMEM((K,), jnp.int32),
            pltpu.VMEM((K,), jnp.int32),
            pltpu.VMEM((K, H), jnp.float32),
            pltpu.VMEM((K, H), jnp.float32),
            pltpu.VMEM((64, H), jnp.float32),
            pltpu.VMEM_SHARED((NPAD, H), jnp.float32),
            pltpu.SemaphoreType.DMA,
            pltpu.SemaphoreType.DMA,
            pltpu.SemaphoreType.DMA,
            pltpu.SemaphoreType.DMA,
        ],
    )(_scatter_body)


def _scatter_body(y_hbm, src_hbm, dst_hbm, ew_hbm, acc_hbm,
                  srcb, ewv, dstbA, dstbB, rowsA, rowsB, zb, acc_sh,
                  gsemA, gsemB, dsemA, dsemB):
    cid = lax.axis_index("c")
    sid = lax.axis_index("s")
    wid = cid * 16 + sid

    def zrow(r, carry):
        for f in range(H // 16):
            zb[r, pl.ds(f * 16, 16)] = jnp.zeros((16,), jnp.float32)
        return carry

    lax.fori_loop(0, 64, zrow, 0)
    for t in range(ROWS_PER_TILE // 64):
        pltpu.sync_copy(zb, acc_sh.at[pl.ds(sid * ROWS_PER_TILE + t * 64, 64)])
    plsc.subcore_barrier()

    gbase = wid * EPT
    pltpu.sync_copy(src_hbm.at[pl.ds(gbase, EPT)], srcb)
    pltpu.sync_copy(ew_hbm.at[pl.ds(gbase, EPT)], ewv)

    def issue(off, rows, dstb, gsem, dsem):
        pltpu.async_copy(y_hbm.at[srcb.at[pl.ds(off, K)]], rows, gsem)
        pltpu.async_copy(dst_hbm.at[pl.ds(gbase + off, K)], dstb, dsem)

    def scale(off, rows):
        for j in range(K):
            s = plsc.load_gather(ewv, [jnp.full((16,), off + j, jnp.int32)])
            for f in range(H // 16):
                rows[j, pl.ds(f * 16, 16)] = rows[j, pl.ds(f * 16, 16)] * s

    def drain(off, rows, dstb, gsem, dsem):
        pltpu.make_async_copy(y_hbm.at[srcb.at[pl.ds(0, K)]], rows,
                              gsem).wait()
        scale(off, rows)
        pltpu.make_async_copy(dst_hbm.at[pl.ds(gbase, K)], dstb, dsem).wait()
        pltpu.sync_copy(rows, acc_sh.at[dstb], add=True)

    issue(0, rowsA, dstbA, gsemA, dsemA)

    def pair(t, carry):
        offA = (2 * t) * K
        issue(offA + K, rowsB, dstbB, gsemB, dsemB)
        drain(offA, rowsA, dstbA, gsemA, dsemA)
        issue(offA + 2 * K, rowsA, dstbA, gsemA, dsemA)
        drain(offA + K, rowsB, dstbB, gsemB, dsemB)
        return carry

    lax.fori_loop(0, (NSTEP - 1) // 2, pair, 0)
    drain((NSTEP - 1) * K, rowsA, dstbA, gsemA, dsemA)

    plsc.subcore_barrier()
    pltpu.sync_copy(
        acc_sh.at[pl.ds(sid * ROWS_PER_TILE, ROWS_PER_TILE)],
        acc_hbm.at[cid, pl.ds(sid * ROWS_PER_TILE, ROWS_PER_TILE)],
    )


BLK = 2048
GRID = NPAD // BLK


def _lrelu(x):
    return jnp.where(x > 0, x, 0.01 * x)


def _dis_block(degp):
    deg = degp[0] + degp[1] + 1.0
    return lax.rsqrt(deg)[:, None]


def _mm1_body(x_ref, w_ref, degp_ref, xw_ref, y_ref):
    xw = jnp.dot(x_ref[...], w_ref[...], preferred_element_type=jnp.float32)
    dis = _dis_block(degp_ref[...])
    xw_ref[...] = xw
    y_ref[...] = xw * dis


def _mm1(X_pad, W1, degp):
    return pl.pallas_call(
        _mm1_body,
        grid=(GRID,),
        in_specs=[
            pl.BlockSpec((BLK, FIN), lambda i: (i, 0)),
            pl.BlockSpec((FIN, H), lambda i: (0, 0)),
            pl.BlockSpec((2, BLK), lambda i: (0, i)),
        ],
        out_specs=[pl.BlockSpec((BLK, H), lambda i: (i, 0))] * 2,
        out_shape=[jax.ShapeDtypeStruct((NPAD, H), jnp.float32)] * 2,
    )(X_pad, W1, degp)


def _mm2_body(accp_ref, xw1_ref, degp_ref, b_ref, w_ref, xw2_ref, y2_ref):
    acc = accp_ref[0] + accp_ref[1]
    dis = _dis_block(degp_ref[...])
    h = dis * acc + (dis * dis) * xw1_ref[...] + b_ref[...]
    h = _lrelu(h)
    xw2 = jnp.dot(h, w_ref[...], preferred_element_type=jnp.float32)
    xw2_ref[...] = xw2
    y2_ref[...] = xw2 * dis


def _mm2(accp, xw1, degp, b1, W2):
    return pl.pallas_call(
        _mm2_body,
        grid=(GRID,),
        in_specs=[
            pl.BlockSpec((2, BLK, H), lambda i: (0, i, 0)),
            pl.BlockSpec((BLK, H), lambda i: (i, 0)),
            pl.BlockSpec((2, BLK), lambda i: (0, i)),
            pl.BlockSpec((1, H), lambda i: (0, 0)),
            pl.BlockSpec((H, H), lambda i: (0, 0)),
        ],
        out_specs=[pl.BlockSpec((BLK, H), lambda i: (i, 0))] * 2,
        out_shape=[jax.ShapeDtypeStruct((NPAD, H), jnp.float32)] * 2,
    )(accp, xw1, degp, b1, W2)


def _head_body(accp_ref, xw2_ref, degp_ref, b2_ref, wm1_ref, bm1_ref,
               wm2_ref, bm2_ref, out_ref):
    acc = accp_ref[0] + accp_ref[1]
    dis = _dis_block(degp_ref[...])
    h = dis * acc + (dis * dis) * xw2_ref[...] + b2_ref[...]
    h = _lrelu(h)
    h = _lrelu(jnp.dot(h, wm1_ref[...], preferred_element_type=jnp.float32)
               + bm1_ref[...])
    logits = jnp.dot(h, wm2_ref[...], preferred_element_type=jnp.float32) \
        + bm2_ref[...]
    m = jnp.max(logits, axis=-1, keepdims=True)
    e = jnp.exp(logits - m)
    out_ref[...] = e / jnp.sum(e, axis=-1, keepdims=True)


def _head(accp, xw2, degp, b2, Wm1, bm1, Wm2, bm2):
    return pl.pallas_call(
        _head_body,
        grid=(GRID,),
        in_specs=[
            pl.BlockSpec((2, BLK, H), lambda i: (0, i, 0)),
            pl.BlockSpec((BLK, H), lambda i: (i, 0)),
            pl.BlockSpec((2, BLK), lambda i: (0, i)),
            pl.BlockSpec((1, H), lambda i: (0, 0)),
            pl.BlockSpec((H, H), lambda i: (0, 0)),
            pl.BlockSpec((1, H), lambda i: (0, 0)),
            pl.BlockSpec((H, C), lambda i: (0, 0)),
            pl.BlockSpec((1, C), lambda i: (0, 0)),
        ],
        out_specs=pl.BlockSpec((BLK, C), lambda i: (i, 0)),
        out_shape=jax.ShapeDtypeStruct((NPAD, C), jnp.float32),
    )(accp, xw2, degp, b2, Wm1, bm1, Wm2, bm2)


def kernel(X, edge_index, edge_weight, W1, b1, W2, b2, Wm1, bm1, Wm2, bm2):
    src = jnp.pad(edge_index[0], (0, EPAD - E))
    dst = jnp.pad(edge_index[1], (0, EPAD - E))
    ew = jnp.pad(edge_weight, (0, EPAD - E))
    X_pad = jnp.pad(X, ((0, NPAD - N), (0, 0)))

    deg_k = _make_deg_kernel()
    scat_k = _make_scatter_kernel()
    degp = deg_k(dst, ew)
    xw1, y1 = _mm1(X_pad, W1, degp)
    acc1 = scat_k(y1, src, dst, ew)
    xw2, y2 = _mm2(acc1, xw1, degp, b1.reshape(1, H), W2)
    acc2 = scat_k(y2, src, dst, ew)
    out = _head(acc2, xw2, degp, b2.reshape(1, H), Wm1, bm1.reshape(1, H),
                Wm2, bm2.reshape(1, C))
    return out[:N]


# speedup vs baseline: 1.1699x; 1.0134x over previous
import functools

import jax
import jax.numpy as jnp
from jax import lax
from jax.experimental import pallas as pl
from jax.experimental.pallas import tpu as pltpu
from jax.experimental.pallas import tpu_sc as plsc

N, E, FIN, H, C = 10000, 320000, 128, 64, 2
NPAD = 10240
NW = 32
ROWS_PER_TILE = NPAD // 16

K = 80
NSTEP = 125
EPT = K * NSTEP

KD = 128
NSTEPD = 79
EPTD = KD * NSTEPD
EPAD = EPTD * NW


@functools.cache
def _make_deg_kernel():
    return functools.partial(
        pl.kernel,
        out_type=jax.ShapeDtypeStruct((2, NPAD), jnp.float32),
        mesh=plsc.VectorSubcoreMesh(core_axis_name="c", subcore_axis_name="s"),
        compiler_params=pltpu.CompilerParams(use_tc_tiling_on_sc=False,
                                             needs_layout_passes=False),
        scratch_types=[
            pltpu.VMEM((KD,), jnp.int32),
            pltpu.VMEM((KD,), jnp.int32),
            pltpu.VMEM((EPTD,), jnp.float32),
            pltpu.VMEM((ROWS_PER_TILE,), jnp.float32),
            pltpu.VMEM_SHARED((NPAD,), jnp.float32),
            pltpu.SemaphoreType.DMA,
            pltpu.SemaphoreType.DMA,
        ],
    )(_deg_body)


def _deg_body(dst_hbm, ew_hbm, degp_hbm, dstbA, dstbB, ewv, zb, acc_sh,
              dsemA, dsemB):
    cid = lax.axis_index("c")
    sid = lax.axis_index("s")
    wid = cid * 16 + sid

    def zrow(r, carry):
        zb[pl.ds(r * 16, 16)] = jnp.zeros((16,), jnp.float32)
        return carry

    lax.fori_loop(0, ROWS_PER_TILE // 16, zrow, 0)
    pltpu.sync_copy(zb, acc_sh.at[pl.ds(sid * ROWS_PER_TILE, ROWS_PER_TILE)])
    plsc.subcore_barrier()

    gbase = wid * EPTD
    pltpu.sync_copy(ew_hbm.at[pl.ds(gbase, EPTD)], ewv)

    def issue(off, dstb, dsem):
        pltpu.async_copy(dst_hbm.at[pl.ds(gbase + off, KD)], dstb, dsem)

    def drain(off, dstb, dsem):
        pltpu.make_async_copy(dst_hbm.at[pl.ds(gbase, KD)], dstb, dsem).wait()
        pltpu.sync_copy(ewv.at[pl.ds(off, KD)], acc_sh.at[dstb], add=True)

    issue(0, dstbA, dsemA)

    def pair(t, carry):
        offA = (2 * t) * KD
        issue(offA + KD, dstbB, dsemB)
        drain(offA, dstbA, dsemA)
        issue(offA + 2 * KD, dstbA, dsemA)
        drain(offA + KD, dstbB, dsemB)
        return carry

    lax.fori_loop(0, (NSTEPD - 1) // 2, pair, 0)
    drain((NSTEPD - 1) * KD, dstbA, dsemA)
    plsc.subcore_barrier()
    pltpu.sync_copy(
        acc_sh.at[pl.ds(sid * ROWS_PER_TILE, ROWS_PER_TILE)],
        degp_hbm.at[cid, pl.ds(sid * ROWS_PER_TILE, ROWS_PER_TILE)],
    )


@functools.cache
def _make_scatter_kernel():
    return functools.partial(
        pl.kernel,
        out_type=jax.ShapeDtypeStruct((2, NPAD, H), jnp.float32),
        mesh=plsc.VectorSubcoreMesh(core_axis_name="c", subcore_axis_name="s"),
        compiler_params=pltpu.CompilerParams(use_tc_tiling_on_sc=False,
                                             needs_layout_passes=False),
        scratch_types=[
            pltpu.VMEM((EPT,), jnp.int32),
            pltpu.VMEM((EPT,), jnp.float32),
            pltpu.VMEM((K,), jnp.int32),
            pltpu.VMEM((K,), jnp.int32),
            pltpu.VMEM((K, H), jnp.float32),
            pltpu.VMEM((K, H), jnp.float32),
            pltpu.VMEM((64, H), jnp.float32),
            pltpu.VMEM_SHARED((NPAD, H), jnp.float32),
            pltpu.SemaphoreType.DMA,
            pltpu.SemaphoreType.DMA,
            pltpu.SemaphoreType.DMA,
            pltpu.SemaphoreType.DMA,
        ],
    )(_scatter_body)


def _scatter_body(y_hbm, src_hbm, dst_hbm, ew_hbm, acc_hbm,
                  srcb, ewv, dstbA, dstbB, rowsA, rowsB, zb, acc_sh,
                  gsemA, gsemB, dsemA, dsemB):
    cid = lax.axis_index("c")
    sid = lax.axis_index("s")
    wid = cid * 16 + sid

    def zrow(r, carry):
        for f in range(H // 16):
            zb[r, pl.ds(f * 16, 16)] = jnp.zeros((16,), jnp.float32)
        return carry

    lax.fori_loop(0, 64, zrow, 0)
    for t in range(ROWS_PER_TILE // 64):
        pltpu.sync_copy(zb, acc_sh.at[pl.ds(sid * ROWS_PER_TILE + t * 64, 64)])
    plsc.subcore_barrier()

    gbase = wid * EPT
    pltpu.sync_copy(src_hbm.at[pl.ds(gbase, EPT)], srcb)
    pltpu.sync_copy(ew_hbm.at[pl.ds(gbase, EPT)], ewv)

    def issue(off, rows, dstb, gsem, dsem):
        pltpu.async_copy(y_hbm.at[srcb.at[pl.ds(off, K)]], rows, gsem)
        pltpu.async_copy(dst_hbm.at[pl.ds(gbase + off, K)], dstb, dsem)

    def scale(off, rows):
        for j in range(K):
            s = plsc.load_gather(ewv, [jnp.full((16,), off + j, jnp.int32)])
            for f in range(H // 16):
                rows[j, pl.ds(f * 16, 16)] = rows[j, pl.ds(f * 16, 16)] * s

    def drain(off, rows, dstb, gsem, dsem):
        pltpu.make_async_copy(y_hbm.at[srcb.at[pl.ds(0, K)]], rows,
                              gsem).wait()
        scale(off, rows)
        pltpu.make_async_copy(dst_hbm.at[pl.ds(gbase, K)], dstb, dsem).wait()
        pltpu.sync_copy(rows, acc_sh.at[dstb], add=True)

    issue(0, rowsA, dstbA, gsemA, dsemA)

    def pair(t, carry):
        offA = (2 * t) * K
        issue(offA + K, rowsB, dstbB, gsemB, dsemB)
        drain(offA, rowsA, dstbA, gsemA, dsemA)
        issue(offA + 2 * K, rowsA, dstbA, gsemA, dsemA)
        drain(offA + K, rowsB, dstbB, gsemB, dsemB)
        return carry

    lax.fori_loop(0, (NSTEP - 1) // 2, pair, 0)
    drain((NSTEP - 1) * K, rowsA, dstbA, gsemA, dsemA)

    plsc.subcore_barrier()
    pltpu.sync_copy(
        acc_sh.at[pl.ds(sid * ROWS_PER_TILE, ROWS_PER_TILE)],
        acc_hbm.at[cid, pl.ds(sid * ROWS_PER_TILE, ROWS_PER_TILE)],
    )


BLK = 5120
GRID = NPAD // BLK


def _lrelu(x):
    return jnp.where(x > 0, x, 0.01 * x)


def _dis_block(degp):
    deg = degp[0] + degp[1] + 1.0
    return lax.rsqrt(deg)[:, None]


def _mm1_body(x_ref, w_ref, degp_ref, xw_ref, y_ref):
    xw = jnp.dot(x_ref[...], w_ref[...], preferred_element_type=jnp.float32)
    dis = _dis_block(degp_ref[...])
    xw_ref[...] = xw
    y_ref[...] = xw * dis


def _mm1(X_pad, W1, degp):
    return pl.pallas_call(
        _mm1_body,
        grid=(GRID,),
        in_specs=[
            pl.BlockSpec((BLK, FIN), lambda i: (i, 0)),
            pl.BlockSpec((FIN, H), lambda i: (0, 0)),
            pl.BlockSpec((2, BLK), lambda i: (0, i)),
        ],
        out_specs=[pl.BlockSpec((BLK, H), lambda i: (i, 0))] * 2,
        out_shape=[jax.ShapeDtypeStruct((NPAD, H), jnp.float32)] * 2,
    )(X_pad, W1, degp)


def _mm2_body(accp_ref, xw1_ref, degp_ref, b_ref, w_ref, xw2_ref, y2_ref):
    acc = accp_ref[0] + accp_ref[1]
    dis = _dis_block(degp_ref[...])
    h = dis * acc + (dis * dis) * xw1_ref[...] + b_ref[...]
    h = _lrelu(h)
    xw2 = jnp.dot(h, w_ref[...], preferred_element_type=jnp.float32)
    xw2_ref[...] = xw2
    y2_ref[...] = xw2 * dis


def _mm2(accp, xw1, degp, b1, W2):
    return pl.pallas_call(
        _mm2_body,
        grid=(GRID,),
        in_specs=[
            pl.BlockSpec((2, BLK, H), lambda i: (0, i, 0)),
            pl.BlockSpec((BLK, H), lambda i: (i, 0)),
            pl.BlockSpec((2, BLK), lambda i: (0, i)),
            pl.BlockSpec((1, H), lambda i: (0, 0)),
            pl.BlockSpec((H, H), lambda i: (0, 0)),
        ],
        out_specs=[pl.BlockSpec((BLK, H), lambda i: (i, 0))] * 2,
        out_shape=[jax.ShapeDtypeStruct((NPAD, H), jnp.float32)] * 2,
    )(accp, xw1, degp, b1, W2)


def _head_body(accp_ref, xw2_ref, degp_ref, b2_ref, wm1_ref, bm1_ref,
               wm2_ref, bm2_ref, out_ref):
    acc = accp_ref[0] + accp_ref[1]
    dis = _dis_block(degp_ref[...])
    h = dis * acc + (dis * dis) * xw2_ref[...] + b2_ref[...]
    h = _lrelu(h)
    h = _lrelu(jnp.dot(h, wm1_ref[...], preferred_element_type=jnp.float32)
               + bm1_ref[...])
    logits = jnp.dot(h, wm2_ref[...], preferred_element_type=jnp.float32) \
        + bm2_ref[...]
    m = jnp.max(logits, axis=-1, keepdims=True)
    e = jnp.exp(logits - m)
    out_ref[...] = e / jnp.sum(e, axis=-1, keepdims=True)


def _head(accp, xw2, degp, b2, Wm1, bm1, Wm2, bm2):
    return pl.pallas_call(
        _head_body,
        grid=(GRID,),
        in_specs=[
            pl.BlockSpec((2, BLK, H), lambda i: (0, i, 0)),
            pl.BlockSpec((BLK, H), lambda i: (i, 0)),
            pl.BlockSpec((2, BLK), lambda i: (0, i)),
            pl.BlockSpec((1, H), lambda i: (0, 0)),
            pl.BlockSpec((H, H), lambda i: (0, 0)),
            pl.BlockSpec((1, H), lambda i: (0, 0)),
            pl.BlockSpec((H, C), lambda i: (0, 0)),
            pl.BlockSpec((1, C), lambda i: (0, 0)),
        ],
        out_specs=pl.BlockSpec((BLK, C), lambda i: (i, 0)),
        out_shape=jax.ShapeDtypeStruct((NPAD, C), jnp.float32),
    )(accp, xw2, degp, b2, Wm1, bm1, Wm2, bm2)


def kernel(X, edge_index, edge_weight, W1, b1, W2, b2, Wm1, bm1, Wm2, bm2):
    src = jnp.pad(edge_index[0], (0, EPAD - E))
    dst = jnp.pad(edge_index[1], (0, EPAD - E))
    ew = jnp.pad(edge_weight, (0, EPAD - E))
    X_pad = jnp.pad(X, ((0, NPAD - N), (0, 0)))

    deg_k = _make_deg_kernel()
    scat_k = _make_scatter_kernel()
    degp = deg_k(dst, ew)
    xw1, y1 = _mm1(X_pad, W1, degp)
    acc1 = scat_k(y1, src, dst, ew)
    xw2, y2 = _mm2(acc1, xw1, degp, b1.reshape(1, H), W2)
    acc2 = scat_k(y2, src, dst, ew)
    out = _head(acc2, xw2, degp, b2.reshape(1, H), Wm1, bm1.reshape(1, H),
                Wm2, bm2.reshape(1, C))
    return out[:N]


# speedup vs baseline: 1.1767x; 1.0058x over previous
import functools

import jax
import jax.numpy as jnp
from jax import lax
from jax.experimental import pallas as pl
from jax.experimental.pallas import tpu as pltpu
from jax.experimental.pallas import tpu_sc as plsc

N, E, FIN, H, C = 10000, 320000, 128, 64, 2
NPAD = 10240
NW = 32
ROWS_PER_TILE = NPAD // 16

K = 80
NSTEP = 125
EPT = K * NSTEP

KD = 128
NSTEPD = 79
EPTD = KD * NSTEPD
EPAD = EPTD * NW


@functools.cache
def _make_deg_kernel():
    return functools.partial(
        pl.kernel,
        out_type=jax.ShapeDtypeStruct((2, NPAD), jnp.float32),
        mesh=plsc.VectorSubcoreMesh(core_axis_name="c", subcore_axis_name="s"),
        compiler_params=pltpu.CompilerParams(use_tc_tiling_on_sc=False,
                                             needs_layout_passes=False),
        scratch_types=[
            pltpu.VMEM((KD,), jnp.int32),
            pltpu.VMEM((KD,), jnp.int32),
            pltpu.VMEM((EPTD,), jnp.float32),
            pltpu.VMEM((ROWS_PER_TILE,), jnp.float32),
            pltpu.VMEM_SHARED((NPAD,), jnp.float32),
            pltpu.SemaphoreType.DMA,
            pltpu.SemaphoreType.DMA,
        ],
    )(_deg_body)


def _deg_body(dst_hbm, ew_hbm, degp_hbm, dstbA, dstbB, ewv, zb, acc_sh,
              dsemA, dsemB):
    cid = lax.axis_index("c")
    sid = lax.axis_index("s")
    wid = cid * 16 + sid

    def zrow(r, carry):
        zb[pl.ds(r * 16, 16)] = jnp.zeros((16,), jnp.float32)
        return carry

    lax.fori_loop(0, ROWS_PER_TILE // 16, zrow, 0)
    pltpu.sync_copy(zb, acc_sh.at[pl.ds(sid * ROWS_PER_TILE, ROWS_PER_TILE)])
    plsc.subcore_barrier()

    gbase = wid * EPTD
    pltpu.sync_copy(ew_hbm.at[pl.ds(gbase, EPTD)], ewv)

    def issue(off, dstb, dsem):
        pltpu.async_copy(dst_hbm.at[pl.ds(gbase + off, KD)], dstb, dsem)

    def drain(off, dstb, dsem):
        pltpu.make_async_copy(dst_hbm.at[pl.ds(gbase, KD)], dstb, dsem).wait()
        pltpu.sync_copy(ewv.at[pl.ds(off, KD)], acc_sh.at[dstb], add=True)

    issue(0, dstbA, dsemA)

    def pair(t, carry):
        offA = (2 * t) * KD
        issue(offA + KD, dstbB, dsemB)
        drain(offA, dstbA, dsemA)
        issue(offA + 2 * KD, dstbA, dsemA)
        drain(offA + KD, dstbB, dsemB)
        return carry

    lax.fori_loop(0, (NSTEPD - 1) // 2, pair, 0)
    drain((NSTEPD - 1) * KD, dstbA, dsemA)
    plsc.subcore_barrier()
    pltpu.sync_copy(
        acc_sh.at[pl.ds(sid * ROWS_PER_TILE, ROWS_PER_TILE)],
        degp_hbm.at[cid, pl.ds(sid * ROWS_PER_TILE, ROWS_PER_TILE)],
    )


@functools.cache
def _make_scatter_kernel():
    return functools.partial(
        pl.kernel,
        out_type=jax.ShapeDtypeStruct((2, NPAD, H), jnp.float32),
        mesh=plsc.VectorSubcoreMesh(core_axis_name="c", subcore_axis_name="s"),
        compiler_params=pltpu.CompilerParams(use_tc_tiling_on_sc=False,
                                             needs_layout_passes=False),
        scratch_types=[
            pltpu.VMEM((EPT,), jnp.int32),
            pltpu.VMEM((EPT,), jnp.float32),
            pltpu.VMEM((ROWS_PER_TILE,), jnp.float32),
            pltpu.VMEM((ROWS_PER_TILE,), jnp.float32),
            pltpu.VMEM((NPAD,), jnp.float32),
            pltpu.VMEM((K,), jnp.int32),
            pltpu.VMEM((K,), jnp.int32),
            pltpu.VMEM((K, H), jnp.float32),
            pltpu.VMEM((K, H), jnp.float32),
            pltpu.VMEM((64, H), jnp.float32),
            pltpu.VMEM_SHARED((NPAD, H), jnp.float32),
            pltpu.VMEM_SHARED((NPAD,), jnp.float32),
            pltpu.SemaphoreType.DMA,
            pltpu.SemaphoreType.DMA,
            pltpu.SemaphoreType.DMA,
            pltpu.SemaphoreType.DMA,
        ],
    )(_scatter_body)


def _scatter_body(y_hbm, src_hbm, dst_hbm, ew_hbm, degp_hbm, acc_hbm,
                  srcb, ewv, dg0, dg1, disv, dstbA, dstbB, rowsA, rowsB,
                  zb, acc_sh, dis_sh, gsemA, gsemB, dsemA, dsemB):
    cid = lax.axis_index("c")
    sid = lax.axis_index("s")
    wid = cid * 16 + sid

    def zrow(r, carry):
        for f in range(H // 16):
            zb[r, pl.ds(f * 16, 16)] = jnp.zeros((16,), jnp.float32)
        return carry

    lax.fori_loop(0, 64, zrow, 0)
    for t in range(ROWS_PER_TILE // 64):
        pltpu.sync_copy(zb, acc_sh.at[pl.ds(sid * ROWS_PER_TILE + t * 64, 64)])

    rbase = sid * ROWS_PER_TILE
    pltpu.sync_copy(degp_hbm.at[0, pl.ds(rbase, ROWS_PER_TILE)], dg0)
    pltpu.sync_copy(degp_hbm.at[1, pl.ds(rbase, ROWS_PER_TILE)], dg1)

    def dis_row(r, carry):
        d = dg0[pl.ds(r * 16, 16)] + dg1[pl.ds(r * 16, 16)] + 1.0
        i = plsc.bitcast(d, jnp.int32)
        y = plsc.bitcast(0x5F3759DF - lax.shift_right_logical(i, 1),
                         jnp.float32)
        for _ in range(3):
            y = y * (1.5 - 0.5 * d * y * y)
        dg0[pl.ds(r * 16, 16)] = y
        return carry

    lax.fori_loop(0, ROWS_PER_TILE // 16, dis_row, 0)
    pltpu.sync_copy(dg0, dis_sh.at[pl.ds(rbase, ROWS_PER_TILE)])
    plsc.subcore_barrier()
    pltpu.sync_copy(dis_sh, disv)

    gbase = wid * EPT
    pltpu.sync_copy(src_hbm.at[pl.ds(gbase, EPT)], srcb)
    pltpu.sync_copy(ew_hbm.at[pl.ds(gbase, EPT)], ewv)

    def coeff_row(r, carry):
        sv = srcb[pl.ds(r * 16, 16)]
        ewv[pl.ds(r * 16, 16)] = ewv[pl.ds(r * 16, 16)] * \
            plsc.load_gather(disv, [sv])
        return carry

    lax.fori_loop(0, EPT // 16, coeff_row, 0)

    def issue(off, rows, dstb, gsem, dsem):
        pltpu.async_copy(y_hbm.at[srcb.at[pl.ds(off, K)]], rows, gsem)
        pltpu.async_copy(dst_hbm.at[pl.ds(gbase + off, K)], dstb, dsem)

    def scale(off, rows):
        for j in range(K):
            s = plsc.load_gather(ewv, [jnp.full((16,), off + j, jnp.int32)])
            for f in range(H // 16):
                rows[j, pl.ds(f * 16, 16)] = rows[j, pl.ds(f * 16, 16)] * s

    def drain(off, rows, dstb, gsem, dsem):
        pltpu.make_async_copy(y_hbm.at[srcb.at[pl.ds(0, K)]], rows,
                              gsem).wait()
        scale(off, rows)
        pltpu.make_async_copy(dst_hbm.at[pl.ds(gbase, K)], dstb, dsem).wait()
        pltpu.sync_copy(rows, acc_sh.at[dstb], add=True)

    issue(0, rowsA, dstbA, gsemA, dsemA)

    def pair(t, carry):
        offA = (2 * t) * K
        issue(offA + K, rowsB, dstbB, gsemB, dsemB)
        drain(offA, rowsA, dstbA, gsemA, dsemA)
        issue(offA + 2 * K, rowsA, dstbA, gsemA, dsemA)
        drain(offA + K, rowsB, dstbB, gsemB, dsemB)
        return carry

    lax.fori_loop(0, (NSTEP - 1) // 2, pair, 0)
    drain((NSTEP - 1) * K, rowsA, dstbA, gsemA, dsemA)

    plsc.subcore_barrier()
    pltpu.sync_copy(
        acc_sh.at[pl.ds(sid * ROWS_PER_TILE, ROWS_PER_TILE)],
        acc_hbm.at[cid, pl.ds(sid * ROWS_PER_TILE, ROWS_PER_TILE)],
    )


BLK = 5120
GRID = NPAD // BLK


def _lrelu(x):
    return jnp.where(x > 0, x, 0.01 * x)


def _dis_block(degp):
    deg = degp[0] + degp[1] + 1.0
    return lax.rsqrt(deg)[:, None]


def _mm1_body(x_ref, w_ref, xw_ref):
    xw_ref[...] = jnp.dot(x_ref[...], w_ref[...],
                          preferred_element_type=jnp.float32)


def _mm1(X_pad, W1):
    return pl.pallas_call(
        _mm1_body,
        grid=(GRID,),
        in_specs=[
            pl.BlockSpec((BLK, FIN), lambda i: (i, 0)),
            pl.BlockSpec((FIN, H), lambda i: (0, 0)),
        ],
        out_specs=pl.BlockSpec((BLK, H), lambda i: (i, 0)),
        out_shape=jax.ShapeDtypeStruct((NPAD, H), jnp.float32),
    )(X_pad, W1)


def _mm2_body(accp_ref, xw1_ref, degp_ref, b_ref, w_ref, xw2_ref):
    acc = accp_ref[0] + accp_ref[1]
    dis = _dis_block(degp_ref[...])
    h = dis * acc + (dis * dis) * xw1_ref[...] + b_ref[...]
    h = _lrelu(h)
    xw2_ref[...] = jnp.dot(h, w_ref[...], preferred_element_type=jnp.float32)


def _mm2(accp, xw1, degp, b1, W2):
    return pl.pallas_call(
        _mm2_body,
        grid=(GRID,),
        in_specs=[
            pl.BlockSpec((2, BLK, H), lambda i: (0, i, 0)),
            pl.BlockSpec((BLK, H), lambda i: (i, 0)),
            pl.BlockSpec((2, BLK), lambda i: (0, i)),
            pl.BlockSpec((1, H), lambda i: (0, 0)),
            pl.BlockSpec((H, H), lambda i: (0, 0)),
        ],
        out_specs=pl.BlockSpec((BLK, H), lambda i: (i, 0)),
        out_shape=jax.ShapeDtypeStruct((NPAD, H), jnp.float32),
    )(accp, xw1, degp, b1, W2)


def _head_body(accp_ref, xw2_ref, degp_ref, b2_ref, wm1_ref, bm1_ref,
               wm2_ref, bm2_ref, out_ref):
    acc = accp_ref[0] + accp_ref[1]
    dis = _dis_block(degp_ref[...])
    h = dis * acc + (dis * dis) * xw2_ref[...] + b2_ref[...]
    h = _lrelu(h)
    h = _lrelu(jnp.dot(h, wm1_ref[...], preferred_element_type=jnp.float32)
               + bm1_ref[...])
    logits = jnp.dot(h, wm2_ref[...], preferred_element_type=jnp.float32) \
        + bm2_ref[...]
    m = jnp.max(logits, axis=-1, keepdims=True)
    e = jnp.exp(logits - m)
    out_ref[...] = e / jnp.sum(e, axis=-1, keepdims=True)


def _head(accp, xw2, degp, b2, Wm1, bm1, Wm2, bm2):
    return pl.pallas_call(
        _head_body,
        grid=(GRID,),
        in_specs=[
            pl.BlockSpec((2, BLK, H), lambda i: (0, i, 0)),
            pl.BlockSpec((BLK, H), lambda i: (i, 0)),
            pl.BlockSpec((2, BLK), lambda i: (0, i)),
            pl.BlockSpec((1, H), lambda i: (0, 0)),
            pl.BlockSpec((H, H), lambda i: (0, 0)),
            pl.BlockSpec((1, H), lambda i: (0, 0)),
            pl.BlockSpec((H, C), lambda i: (0, 0)),
            pl.BlockSpec((1, C), lambda i: (0, 0)),
        ],
        out_specs=pl.BlockSpec((BLK, C), lambda i: (i, 0)),
        out_shape=jax.ShapeDtypeStruct((NPAD, C), jnp.float32),
    )(accp, xw2, degp, b2, Wm1, bm1, Wm2, bm2)


def kernel(X, edge_index, edge_weight, W1, b1, W2, b2, Wm1, bm1, Wm2, bm2):
    src = jnp.pad(edge_index[0], (0, EPAD - E))
    dst = jnp.pad(edge_index[1], (0, EPAD - E))
    ew = jnp.pad(edge_weight, (0, EPAD - E))
    X_pad = jnp.pad(X, ((0, NPAD - N), (0, 0)))

    deg_k = _make_deg_kernel()
    scat_k = _make_scatter_kernel()
    degp = deg_k(dst, ew)
    xw1 = _mm1(X_pad, W1)
    acc1 = scat_k(xw1, src, dst, ew, degp)
    xw2 = _mm2(acc1, xw1, degp, b1.reshape(1, H), W2)
    acc2 = scat_k(xw2, src, dst, ew, degp)
    out = _head(acc2, xw2, degp, b2.reshape(1, H), Wm1, bm1.reshape(1, H),
                Wm2, bm2.reshape(1, C))
    return out[:N]
